# baseline (speedup 1.0000x reference)
import functools

import jax
import jax.numpy as jnp
from jax.experimental import pallas as pl
from jax.experimental.pallas import tpu as pltpu


def _rpq_kernel(x_ref, rp_ref, cb_ref, out_ref, *, k_chunk):
    t = x_ref.shape[0]
    k, e = cb_ref.shape
    proj = jnp.dot(x_ref[...], rp_ref[...],
                   preferred_element_type=jnp.float32)
    nsq = jnp.sum(proj * proj, axis=1, keepdims=True)
    norm = jnp.sqrt(nsq)
    inv = 1.0 / jnp.maximum(norm, 1e-12)
    nx = proj * inv
    b2 = jnp.sum(nx * nx, axis=1, keepdims=True)
    nx_aug = jnp.concatenate([nx, jnp.ones((t, 1), jnp.float32), b2],
                             axis=1)
    nx_augT = nx_aug.T

    def body(j, carry):
        best, bidx = carry
        cbc = cb_ref[pl.ds(j * k_chunk, k_chunk), :]
        a2c = jnp.sum(cbc * cbc, axis=1, keepdims=True)
        wc = jnp.concatenate(
            [cbc * -2.0, a2c, jnp.ones((k_chunk, 1), jnp.float32)],
            axis=1)
        d2 = jnp.dot(wc, nx_augT, preferred_element_type=jnp.float32)
        cmin = jnp.min(d2, axis=0, keepdims=True)
        iota = jax.lax.broadcasted_iota(jnp.int32, d2.shape, 0)
        carg = jnp.min(jnp.where(d2 == cmin, iota, k), axis=0,
                       keepdims=True) + j * k_chunk
        take = cmin < best
        return jnp.where(take, cmin, best), jnp.where(take, carg, bidx)

    best0 = jnp.full((1, t), jnp.inf, jnp.float32)
    bidx0 = jnp.zeros((1, t), jnp.int32)
    _, bidx = jax.lax.fori_loop(0, k // k_chunk, body, (best0, bidx0))
    out_ref[...] = bidx[None]


@jax.jit
def kernel(x, random_projection, codebook):
    b, n, d = x.shape
    bn = b * n
    k, e = codebook.shape
    T = 1024
    KC = 4096
    flat = x.reshape(bn, d)
    out = pl.pallas_call(
        functools.partial(_rpq_kernel, k_chunk=KC),
        grid=(bn // T,),
        in_specs=[
            pl.BlockSpec((T, d), lambda i: (i, 0)),
            pl.BlockSpec((d, e), lambda i: (0, 0)),
            pl.BlockSpec((k, e), lambda i: (0, 0)),
        ],
        out_specs=pl.BlockSpec((1, 1, T), lambda i: (i, 0, 0)),
        out_shape=jax.ShapeDtypeStruct((bn // T, 1, T), jnp.int32),
        compiler_params=pltpu.CompilerParams(
            dimension_semantics=("parallel",)),
    )(flat, random_projection, codebook)
    return out.reshape(b, n)


# speedup vs baseline: 1.0856x; 1.0856x over previous
import functools

import jax
import jax.numpy as jnp
from jax.experimental import pallas as pl
from jax.experimental.pallas import tpu as pltpu


def _rpq_kernel(x_ref, rp_ref, cb_ref, out_ref, *, k_chunk):
    t = x_ref.shape[0]
    k, e = cb_ref.shape
    proj = jnp.dot(x_ref[...], rp_ref[...],
                   preferred_element_type=jnp.float32)
    nsq = jnp.sum(proj * proj, axis=1, keepdims=True)
    norm = jnp.sqrt(nsq)
    inv = 1.0 / jnp.maximum(norm, 1e-12)
    nx = proj * inv
    b2 = jnp.sum(nx * nx, axis=1, keepdims=True)
    nx_aug = jnp.concatenate([nx, jnp.ones((t, 1), jnp.float32), b2],
                             axis=1)
    nx_augT = nx_aug.T

    def body(j, carry):
        best, bidx = carry
        cbc = cb_ref[pl.ds(j * k_chunk, k_chunk), :]
        a2c = jnp.sum(cbc * cbc, axis=1, keepdims=True)
        wc = jnp.concatenate(
            [cbc * -2.0, a2c, jnp.ones((k_chunk, 1), jnp.float32)],
            axis=1)
        d2 = jnp.dot(wc, nx_augT, preferred_element_type=jnp.float32)
        cmin = jnp.min(d2, axis=0, keepdims=True)
        iota = jax.lax.broadcasted_iota(jnp.int32, d2.shape, 0)
        carg = jnp.min(jnp.where(d2 == cmin, iota, k), axis=0,
                       keepdims=True) + j * k_chunk
        take = cmin < best
        return jnp.where(take, cmin, best), jnp.where(take, carg, bidx)

    best0 = jnp.full((1, t), jnp.inf, jnp.float32)
    bidx0 = jnp.zeros((1, t), jnp.int32)
    _, bidx = jax.lax.fori_loop(0, k // k_chunk, body, (best0, bidx0))
    out_ref[...] = bidx[None]


@jax.jit
def kernel(x, random_projection, codebook):
    b, n, d = x.shape
    bn = b * n
    k, e = codebook.shape
    T = 2048
    KC = 2048
    flat = x.reshape(bn, d)
    out = pl.pallas_call(
        functools.partial(_rpq_kernel, k_chunk=KC),
        grid=(bn // T,),
        in_specs=[
            pl.BlockSpec((T, d), lambda i: (i, 0)),
            pl.BlockSpec((d, e), lambda i: (0, 0)),
            pl.BlockSpec((k, e), lambda i: (0, 0)),
        ],
        out_specs=pl.BlockSpec((1, 1, T), lambda i: (i, 0, 0)),
        out_shape=jax.ShapeDtypeStruct((bn // T, 1, T), jnp.int32),
        compiler_params=pltpu.CompilerParams(
            dimension_semantics=("parallel",)),
    )(flat, random_projection, codebook)
    return out.reshape(b, n)
